# baseline (speedup 1.0000x reference)
import functools

import jax
import jax.numpy as jnp
from jax import lax
from jax.experimental import pallas as pl
from jax.experimental.pallas import tpu as pltpu
from jax.experimental.pallas import tpu_sc as plsc

_B = 16384
_F = 39
_NCAT = 26
_NCONT = _F - _NCAT
_V = 100000
_D = 16
_EPS = 1e-5
_OUTW = _NCAT * _D + _NCONT

_NW = 32
_BCH = 4096
_NBCH = _B // _BCH
_UNROLL = 8


def _prelude_body(xT_ref, gamma_ref, beta_ref, contT_ref):
    xc = xT_ref[_NCAT:, :]
    mean = jnp.mean(xc, axis=1, keepdims=True)
    var = jnp.mean((xc - mean) ** 2, axis=1, keepdims=True)
    inv = lax.rsqrt(var + _EPS)
    contT_ref[...] = (xc - mean) * inv * gamma_ref[...] + beta_ref[...]


_QS = 25088
_Q3 = 99968 - 3 * _QS
_NTAIL = 128


def _sc_body(xT, tablesT, tails, contT, outT, row_v, idx_v, outb_v, sem_r, sem_i, sem_o):
    wid = lax.axis_index("s") * 2 + lax.axis_index("c")
    start = wid * 13

    def cat_task(t, f, prev_f):
        d = t & 15
        cps = []
        for q in range(4):
            off = q * _QS
            sz = _QS if q < 3 else _Q3
            cps.append(
                pltpu.async_copy(
                    tablesT.at[f, d, pl.ds(off, sz)],
                    row_v.at[pl.ds(off, sz)],
                    sem_r,
                )
            )
        cps.append(
            pltpu.async_copy(
                tails.at[pl.ds(t * _NTAIL, _NTAIL)],
                row_v.at[pl.ds(_V - _NTAIL, _NTAIL)],
                sem_r,
            )
        )

        @pl.when(f != prev_f)
        def _():
            def conv_chunk(c, carry2):
                b0 = c * _BCH
                pltpu.sync_copy(xT.at[f, pl.ds(b0, _BCH)], outb_v.at[0])

                def conv_body(g, carry3):
                    for j in range(_UNROLL):
                        o = (g * _UNROLL + j) * 16
                        idx_v[pl.ds(b0 + o, 16)] = outb_v[0, pl.ds(o, 16)].astype(
                            jnp.int32
                        )
                    return carry3

                lax.fori_loop(0, _BCH // 16 // _UNROLL, conv_body, 0)
                return carry2

            lax.fori_loop(0, _NBCH, conv_chunk, 0)

        for cp in cps:
            cp.wait()
        cp_o = [None, None]
        for c in range(_NBCH):
            buf = c % 2
            if cp_o[buf] is not None:
                cp_o[buf].wait()
            b0 = c * _BCH

            def g_body(g, carry3):
                for j in range(_UNROLL):
                    o = (g * _UNROLL + j) * 16
                    idx16 = idx_v[pl.ds(b0 + o, 16)]
                    outb_v[buf, pl.ds(o, 16)] = plsc.load_gather(row_v, [idx16])
                return carry3

            lax.fori_loop(0, _BCH // 16 // _UNROLL, g_body, 0)
            cp_o[buf] = pltpu.async_copy(
                outb_v.at[buf], outT.at[t, pl.ds(b0, _BCH)], sem_o
            )
        for cp in cp_o:
            cp.wait()

    def cont_task(t):
        pltpu.sync_copy(contT.at[t - _NCAT * _D, :], row_v.at[pl.ds(0, _B)])
        pltpu.sync_copy(row_v.at[pl.ds(0, _B)], outT.at[t, :])

    def task_body(k, prev_f):
        t = start + k
        f = t >> 4
        cat_task(t, f, prev_f)
        return f

    @pl.when(wid < _NCONT)
    def _():
        cont_task(_NCAT * _D + wid)

    lax.fori_loop(0, 13, task_body, jnp.int32(-1))


@jax.jit
def kernel(x, tables, gamma, beta):
    xT = x.T
    tablesT = jnp.transpose(tables, (0, 2, 1))
    tails = jnp.transpose(tables[:, _V - 128 :, :], (0, 2, 1)).reshape(-1)

    contT = pl.pallas_call(
        _prelude_body,
        out_shape=jax.ShapeDtypeStruct((_NCONT, _B), jnp.float32),
    )(xT, gamma.reshape(_NCONT, 1), beta.reshape(_NCONT, 1))

    sc_call = pl.kernel(
        _sc_body,
        out_type=jax.ShapeDtypeStruct((_OUTW, _B), jnp.float32),
        mesh=plsc.VectorSubcoreMesh(core_axis_name="c", subcore_axis_name="s"),
        scratch_types=[
            pltpu.VMEM((_V,), jnp.float32),
            pltpu.VMEM((_B,), jnp.int32),
            pltpu.VMEM((2, _BCH), jnp.float32),
            pltpu.SemaphoreType.DMA,
            pltpu.SemaphoreType.DMA,
            pltpu.SemaphoreType.DMA,
        ],
        compiler_params=pltpu.CompilerParams(
            use_tc_tiling_on_sc=True, needs_layout_passes=False
        ),
    )
    outT = sc_call(xT, tablesT, tails, contT)
    return outT.T


# speedup vs baseline: 1.0071x; 1.0071x over previous
import functools

import jax
import jax.numpy as jnp
from jax import lax
from jax.experimental import pallas as pl
from jax.experimental.pallas import tpu as pltpu
from jax.experimental.pallas import tpu_sc as plsc

_B = 16384
_F = 39
_NCAT = 26
_NCONT = _F - _NCAT
_V = 100000
_D = 16
_EPS = 1e-5
_OUTW = _NCAT * _D + _NCONT

_NW = 32
_BCH = 2048
_NBCH = _B // _BCH
_UNROLL = 8


def _prelude_body(xT_ref, gamma_ref, beta_ref, contT_ref):
    xc = xT_ref[_NCAT:, :]
    mean = jnp.mean(xc, axis=1, keepdims=True)
    var = jnp.mean((xc - mean) ** 2, axis=1, keepdims=True)
    inv = lax.rsqrt(var + _EPS)
    contT_ref[...] = (xc - mean) * inv * gamma_ref[...] + beta_ref[...]


def _sc_body(xT, tablesT, contT, outT, row_v, idx_v, outb_v, sem_r, sem_i, sem_o):
    wid = lax.axis_index("s") * 2 + lax.axis_index("c")
    start = wid * 13

    def cat_task(t, f, prev_f):
        cp_row = pltpu.async_copy(tablesT.at[f, t & 15, :], row_v, sem_r)

        @pl.when(f != prev_f)
        def _():
            def conv_chunk(c, carry2):
                b0 = c * _BCH
                pltpu.sync_copy(xT.at[f, pl.ds(b0, _BCH)], outb_v.at[0])

                def conv_body(g, carry3):
                    for j in range(_UNROLL):
                        o = (g * _UNROLL + j) * 16
                        idx_v[pl.ds(b0 + o, 16)] = outb_v[0, pl.ds(o, 16)].astype(
                            jnp.int32
                        )
                    return carry3

                lax.fori_loop(0, _BCH // 16 // _UNROLL, conv_body, 0)
                return carry2

            lax.fori_loop(0, _NBCH, conv_chunk, 0)

        cp_row.wait()
        cp_o = [None, None, None, None]
        for c in range(_NBCH):
            buf = c % 4
            if cp_o[buf] is not None:
                cp_o[buf].wait()
            b0 = c * _BCH

            def g_body(g, carry3):
                for j in range(_UNROLL):
                    o = (g * _UNROLL + j) * 16
                    idx16 = idx_v[pl.ds(b0 + o, 16)]
                    outb_v[buf, pl.ds(o, 16)] = plsc.load_gather(row_v, [idx16])
                return carry3

            lax.fori_loop(0, _BCH // 16 // _UNROLL, g_body, 0)
            cp_o[buf] = pltpu.async_copy(
                outb_v.at[buf], outT.at[t, pl.ds(b0, _BCH)], sem_o
            )
        for cp in cp_o:
            cp.wait()

    def cont_task(t):
        pltpu.sync_copy(contT.at[t - _NCAT * _D, :], row_v.at[pl.ds(0, _B)])
        pltpu.sync_copy(row_v.at[pl.ds(0, _B)], outT.at[t, :])

    def task_body(k, prev_f):
        t = start + k
        f = t >> 4
        cat_task(t, f, prev_f)
        return f

    lax.fori_loop(0, 13, task_body, jnp.int32(-1))

    @pl.when(wid < _NCONT)
    def _():
        cont_task(_NCAT * _D + wid)


@jax.jit
def kernel(x, tables, gamma, beta):
    xT = x.T
    tablesT = jnp.transpose(tables, (0, 2, 1))

    contT = pl.pallas_call(
        _prelude_body,
        out_shape=jax.ShapeDtypeStruct((_NCONT, _B), jnp.float32),
    )(xT, gamma.reshape(_NCONT, 1), beta.reshape(_NCONT, 1))

    sc_call = pl.kernel(
        _sc_body,
        out_type=jax.ShapeDtypeStruct((_OUTW, _B), jnp.float32),
        mesh=plsc.VectorSubcoreMesh(core_axis_name="c", subcore_axis_name="s"),
        scratch_types=[
            pltpu.VMEM((_V,), jnp.float32),
            pltpu.VMEM((_B,), jnp.int32),
            pltpu.VMEM((4, _BCH), jnp.float32),
            pltpu.SemaphoreType.DMA,
            pltpu.SemaphoreType.DMA,
            pltpu.SemaphoreType.DMA,
        ],
        compiler_params=pltpu.CompilerParams(
            use_tc_tiling_on_sc=True, needs_layout_passes=False
        ),
    )
    outT = sc_call(xT, tablesT, contT)
    return outT.T
